# baseline (speedup 1.0000x reference)
import functools

import jax
import jax.numpy as jnp
from jax import lax
from jax.experimental import pallas as pl
from jax.experimental.pallas import tpu as pltpu
from jax.experimental.pallas import tpu_sc as plsc

BATCH = 16384
EMB = 64

try:
    _INFO = plsc.get_sparse_core_info()
    _NC, _NS = _INFO.num_cores, _INFO.num_subcores
except ValueError:
    _NC, _NS = 2, 16
_NW = _NC * _NS
_BPW = BATCH // _NW
_CHN = _BPW // 4


def _sc_gather(user_ids, item_ids, ue_gmf, ie_gmf, ue_mlp, ie_mlp):
    mesh = plsc.VectorSubcoreMesh(core_axis_name="c", subcore_axis_name="s")

    @functools.partial(
        pl.kernel,
        out_type=[jax.ShapeDtypeStruct((BATCH, EMB), jnp.float32)] * 3,
        mesh=mesh,
        scratch_types=[
            pltpu.VMEM((_BPW,), jnp.int32),
            pltpu.VMEM((_BPW,), jnp.int32),
            pltpu.VMEM((_CHN, EMB), jnp.float32),
            pltpu.VMEM((_CHN, EMB), jnp.float32),
            pltpu.VMEM((_CHN, EMB), jnp.float32),
            pltpu.VMEM((_CHN, EMB), jnp.float32),
            pltpu.SemaphoreType.DMA,
        ],
    )
    def k(uids, iids, ueg, ieg, uem, iem, out_gmf, out_um, out_im,
          uidx, iidx, bug, big, bum, bim, sem):
        wid = lax.axis_index("s") * _NC + lax.axis_index("c")
        base = wid * _BPW
        pltpu.sync_copy(uids.at[pl.ds(base, _BPW)], uidx)
        pltpu.sync_copy(iids.at[pl.ds(base, _BPW)], iidx)

        for h in range(4):
            off = h * _CHN

            def fire(g, _):
                uvec = uidx[pl.ds(off + g * 16, 16)]
                ivec = iidx[pl.ds(off + g * 16, 16)]
                for kk in range(16):
                    u = uvec[kk]
                    i = ivec[kk]
                    j = g * 16 + kk
                    pltpu.async_copy(ueg.at[u], bug.at[j], sem)
                    pltpu.async_copy(ieg.at[i], big.at[j], sem)
                    pltpu.async_copy(uem.at[u], bum.at[j], sem)
                    pltpu.async_copy(iem.at[i], bim.at[j], sem)
                return _

            lax.fori_loop(0, _CHN // 16, fire, 0)
            for buf in (bug, big, bum, bim):
                pltpu.make_async_copy(ueg.at[pl.ds(0, _CHN)], buf, sem).wait()

            def mul(j, _):
                for c in range(EMB // 16):
                    s = pl.ds(c * 16, 16)
                    bug[j, s] = bug[j, s] * big[j, s]
                return _

            lax.fori_loop(0, _CHN, mul, 0)
            pltpu.sync_copy(bug, out_gmf.at[pl.ds(base + off, _CHN)])
            pltpu.sync_copy(bum, out_um.at[pl.ds(base + off, _CHN)])
            pltpu.sync_copy(bim, out_im.at[pl.ds(base + off, _CHN)])

    return k(user_ids, item_ids, ue_gmf, ie_gmf, ue_mlp, ie_mlp)


_BLK = 2048


def _mlp_body(gmf, um, im, w1u, w1i, b1, w2, b2, w3, b3, wog, wom, bo, out):
    h = jnp.dot(um[...], w1u[...], preferred_element_type=jnp.float32)
    h = h + jnp.dot(im[...], w1i[...], preferred_element_type=jnp.float32)
    h = jax.nn.relu(h + b1[...])
    h = jax.nn.relu(jnp.dot(h, w2[...], preferred_element_type=jnp.float32) + b2[...])
    h = jax.nn.relu(jnp.dot(h, w3[...], preferred_element_type=jnp.float32) + b3[...])
    o = jnp.dot(gmf[...], wog[...], preferred_element_type=jnp.float32)
    o = o + jnp.dot(h, wom[...], preferred_element_type=jnp.float32)
    out[...] = o + bo[...]


def _tc_mlp(gmf, um, im, W1, b1, W2, b2, W3, b3, Wo, bo):
    grid = (BATCH // _BLK,)
    bspec = pl.BlockSpec((_BLK, EMB), lambda i: (i, 0))

    def whole(shape):
        return pl.BlockSpec(shape, lambda i: (0,) * len(shape))

    return pl.pallas_call(
        _mlp_body,
        grid=grid,
        in_specs=[bspec, bspec, bspec,
                  whole((EMB, 128)), whole((EMB, 128)), whole((1, 128)),
                  whole((128, 64)), whole((1, 64)),
                  whole((64, 32)), whole((1, 32)),
                  whole((EMB, 1)), whole((32, 1)), whole((1, 1))],
        out_specs=pl.BlockSpec((_BLK, 1), lambda i: (i, 0)),
        out_shape=jax.ShapeDtypeStruct((BATCH, 1), jnp.float32),
    )(gmf, um, im, W1[:EMB], W1[EMB:], b1.reshape(1, -1),
      W2, b2.reshape(1, -1), W3, b3.reshape(1, -1),
      Wo[:EMB], Wo[EMB:], bo.reshape(1, -1))


def kernel(user_ids, item_ids, ue_gmf, ie_gmf, ue_mlp, ie_mlp,
           W1, b1, W2, b2, W3, b3, Wo, bo):
    user_ids = user_ids.astype(jnp.int32)
    item_ids = item_ids.astype(jnp.int32)
    gmf, um, im = _sc_gather(user_ids, item_ids, ue_gmf, ie_gmf,
                             ue_mlp, ie_mlp)
    return _tc_mlp(gmf, um, im, W1, b1, W2, b2, W3, b3, Wo, bo)


# speedup vs baseline: 1.0296x; 1.0296x over previous
import functools

import jax
import jax.numpy as jnp
from jax import lax
from jax.experimental import pallas as pl
from jax.experimental.pallas import tpu as pltpu
from jax.experimental.pallas import tpu_sc as plsc

BATCH = 16384
EMB = 64

try:
    _INFO = plsc.get_sparse_core_info()
    _NC, _NS = _INFO.num_cores, _INFO.num_subcores
except ValueError:
    _NC, _NS = 2, 16
_NW = _NC * _NS
_BPW = BATCH // _NW
_CHN = _BPW // 4


def _sc_gather(user_ids, item_ids, ue_gmf, ie_gmf, ue_mlp, ie_mlp):
    mesh = plsc.VectorSubcoreMesh(core_axis_name="c", subcore_axis_name="s")

    @functools.partial(
        pl.kernel,
        out_type=[jax.ShapeDtypeStruct((BATCH, EMB), jnp.float32)] * 3,
        mesh=mesh,
        scratch_types=[
            pltpu.VMEM((_BPW,), jnp.int32),
            pltpu.VMEM((_BPW,), jnp.int32),
            pltpu.VMEM((_CHN, EMB), jnp.float32),
            pltpu.VMEM((_CHN, EMB), jnp.float32),
            pltpu.VMEM((_CHN, EMB), jnp.float32),
            pltpu.VMEM((_CHN, EMB), jnp.float32),
            pltpu.SemaphoreType.DMA,
        ],
    )
    def k(uids, iids, ueg, ieg, uem, iem, out_gmf, out_um, out_im,
          uidx, iidx, bug, big, bum, bim, sem):
        wid = lax.axis_index("s") * _NC + lax.axis_index("c")
        base = wid * _BPW
        pltpu.sync_copy(uids.at[pl.ds(base, _BPW)], uidx)
        pltpu.sync_copy(iids.at[pl.ds(base, _BPW)], iidx)

        for h in range(4):
            off = h * _CHN

            def fire(g, _):
                uvec = uidx[pl.ds(off + g * 16, 16)]
                ivec = iidx[pl.ds(off + g * 16, 16)]
                for kk in range(16):
                    u = uvec[kk]
                    i = ivec[kk]
                    j = g * 16 + kk
                    pltpu.async_copy(ueg.at[u], bug.at[j], sem)
                    pltpu.async_copy(ieg.at[i], big.at[j], sem)
                    pltpu.async_copy(uem.at[u], bum.at[j], sem)
                    pltpu.async_copy(iem.at[i], bim.at[j], sem)
                return _

            lax.fori_loop(0, _CHN // 16, fire, 0)
            for buf in (bug, big, bum, bim):
                pltpu.make_async_copy(ueg.at[pl.ds(0, _CHN)], buf, sem).wait()

            def mul(j, _):
                for c in range(EMB // 16):
                    s = pl.ds(c * 16, 16)
                    bug[j, s] = bug[j, s] * big[j, s]
                return _

            lax.fori_loop(0, _CHN, mul, 0)
            pltpu.sync_copy(bug, out_gmf.at[pl.ds(base + off, _CHN)])
            pltpu.sync_copy(bum, out_um.at[pl.ds(base + off, _CHN)])
            pltpu.sync_copy(bim, out_im.at[pl.ds(base + off, _CHN)])

    return k(user_ids, item_ids, ue_gmf, ie_gmf, ue_mlp, ie_mlp)


_BLK = 2048


def _mlp_body(gmf, um, im, w1u, w1i, b1, w2, b2, w3, b3, wog, wom, bo, out):
    h = jnp.dot(um[...], w1u[...], preferred_element_type=jnp.float32)
    h = h + jnp.dot(im[...], w1i[...], preferred_element_type=jnp.float32)
    h = jax.nn.relu(h + b1[...])
    h = jax.nn.relu(jnp.dot(h, w2[...], preferred_element_type=jnp.float32) + b2[...])
    h = jax.nn.relu(jnp.dot(h, w3[...], preferred_element_type=jnp.float32) + b3[...])
    o = jnp.dot(gmf[...], wog[...], preferred_element_type=jnp.float32)
    o = o + jnp.dot(h, wom[...], preferred_element_type=jnp.float32)
    out[...] = o + bo[...]


def _tc_mlp(gmf, um, im, W1, b1, W2, b2, W3, b3, Wo, bo):
    grid = (BATCH // _BLK,)
    bspec = pl.BlockSpec((_BLK, EMB), lambda i: (i, 0))

    def whole(shape):
        return pl.BlockSpec(shape, lambda i: (0,) * len(shape))

    return pl.pallas_call(
        _mlp_body,
        grid=grid,
        in_specs=[bspec, bspec, bspec,
                  whole((EMB, 128)), whole((EMB, 128)), whole((1, 128)),
                  whole((128, 64)), whole((1, 64)),
                  whole((64, 32)), whole((1, 32)),
                  whole((EMB, 1)), whole((32, 1)), whole((1, 1))],
        out_specs=pl.BlockSpec((_BLK, 1), lambda i: (i, 0)),
        out_shape=jax.ShapeDtypeStruct((BATCH, 1), jnp.float32),
    )(gmf, um, im, W1[:EMB], W1[EMB:], b1.reshape(1, -1),
      W2, b2.reshape(1, -1), W3, b3.reshape(1, -1),
      Wo[:EMB], Wo[EMB:], bo.reshape(1, -1))


def kernel(user_ids, item_ids, ue_gmf, ie_gmf, ue_mlp, ie_mlp,
           W1, b1, W2, b2, W3, b3, Wo, bo):
    user_ids = user_ids.astype(jnp.int32)
    item_ids = item_ids.astype(jnp.int32)
    gmf, um, im = _sc_gather(user_ids, item_ids, ue_gmf, ie_gmf,
                             ue_mlp, ie_mlp)
    h = jax.nn.relu(um @ W1[:EMB] + im @ W1[EMB:] + b1)
    h = jax.nn.relu(h @ W2 + b2)
    h = jax.nn.relu(h @ W3 + b3)
    return gmf @ Wo[:EMB] + h @ Wo[EMB:] + bo
